# Initial kernel scaffold; baseline (speedup 1.0000x reference)
import jax
import jax.numpy as jnp
from jax.experimental import pallas as pl


def kernel(atom_type, pos, bond_index, bond_type, batch, time_step, edge_index, edge_type, edge_length, alphas, emb_g, Wg_a, bg_a, Wg_b, bg_b, emb_l, Wl_a, bl_a, Wl_b, bl_b, Wg1, bg1, Wg2, bg2, Wg3, bg3, Wl1, bl1, Wl2, bl2, Wl3, bl3):
    raise NotImplementedError("write your pallas kernel here")


# speedup vs baseline: 2.8016x; 2.8016x over previous
import functools

import jax
import jax.numpy as jnp
from jax import lax
from jax.experimental import pallas as pl
from jax.experimental.pallas import tpu as pltpu
from jax.experimental.pallas import tpu_sc as plsc

F32 = jnp.float32


def _enc_body(el_ref, et_ref, wga_ref, bga_ref, wgb_ref, bgb_ref, embg_ref,
              wla_ref, bla_ref, wlb_ref, blb_ref, embl_ref,
              outg_ref, outl_ref):
    el = el_ref[...]
    et = et_ref[...]
    eb = el.shape[0]
    ncls = embg_ref.shape[0]
    oh = (et == lax.broadcasted_iota(jnp.int32, (eb, ncls), 1)).astype(F32)

    def enc(wa, ba, wb, bb, emb):
        h = jnp.maximum(el * wa + ba, 0.0)
        d = jnp.dot(h, wb, preferred_element_type=F32) + bb
        return d * jnp.dot(oh, emb, preferred_element_type=F32)

    outg_ref[...] = enc(wga_ref[...], bga_ref[...], wgb_ref[...],
                        bgb_ref[...], embg_ref[...])
    outl_ref[...] = enc(wla_ref[...], bla_ref[...], wlb_ref[...],
                        blb_ref[...], embl_ref[...])


def _edge_encode(el, et2, Wg_a, bg_a, Wg_b, bg_b, emb_g,
                 Wl_a, bl_a, Wl_b, bl_b, emb_l, eb=2000):
    E = el.shape[0]
    H = emb_g.shape[1]
    C = emb_g.shape[0]
    full = lambda s: pl.BlockSpec(s, lambda i: (0, 0))
    return pl.pallas_call(
        _enc_body,
        grid=(E // eb,),
        in_specs=[
            pl.BlockSpec((eb, 1), lambda i: (i, 0)),
            pl.BlockSpec((eb, 1), lambda i: (i, 0)),
            full((1, H)), full((1, H)), full((H, H)), full((1, H)),
            full((C, H)),
            full((1, H)), full((1, H)), full((H, H)), full((1, H)),
            full((C, H)),
        ],
        out_specs=[pl.BlockSpec((eb, H), lambda i: (i, 0))] * 2,
        out_shape=[jax.ShapeDtypeStruct((E, H), F32)] * 2,
    )(el, et2, Wg_a, bg_a, Wg_b, bg_b, emb_g,
      Wl_a, bl_a, Wl_b, bl_b, emb_l)


def _sc_scatter(attr_g, attr_l, col, N):
    E, H = attr_g.shape
    info = plsc.get_sparse_core_info()
    ns = info.num_subcores
    ept = E // ns
    ch = 80
    nch = ept // ch
    rpb = (N // (8 * ns)) * 8
    tail = N - rpb * ns
    zb = 104
    nz = rpb // zb
    zrem = rpb - nz * zb
    mesh = plsc.VectorSubcoreMesh(core_axis_name="c", subcore_axis_name="s")

    @functools.partial(
        pl.kernel,
        out_type=(jax.ShapeDtypeStruct((N, H), F32),
                  jax.ShapeDtypeStruct((N, H), F32)),
        mesh=mesh,
        scratch_types=[
            pltpu.VMEM((ch,), jnp.int32),
            pltpu.VMEM((ch, H), F32),
            pltpu.VMEM((zb, H), F32),
            pltpu.VMEM_SHARED((N, H), F32),
        ],
    )
    def scat(attrg_h, attrl_h, col_h, outg_h, outl_h,
             idx_v, rows_v, zb_v, tab_s):
        cid = lax.axis_index("c")
        sid = lax.axis_index("s")

        def zrow(r, carry):
            for j in range(H // 16):
                zb_v[r, pl.ds(j * 16, 16)] = jnp.zeros((16,), F32)
            return carry
        lax.fori_loop(0, zb, zrow, 0)

        def zcp(k, carry):
            pltpu.sync_copy(zb_v, tab_s.at[pl.ds(sid * rpb + k * zb, zb)])
            return carry
        lax.fori_loop(0, nz, zcp, 0)
        if zrem:
            pltpu.sync_copy(zb_v.at[pl.ds(0, zrem)],
                            tab_s.at[pl.ds(sid * rpb + nz * zb, zrem)])
        if tail:
            @pl.when(sid == ns - 1)
            def _():
                pltpu.sync_copy(zb_v.at[pl.ds(0, tail)],
                                tab_s.at[pl.ds(ns * rpb, tail)])
        plsc.subcore_barrier()

        def make_body(attr_h):
            def body(i, carry):
                off = sid * ept + i * ch
                pltpu.sync_copy(col_h.at[pl.ds(off, ch)], idx_v)
                pltpu.sync_copy(attr_h.at[pl.ds(off, ch)], rows_v)
                pltpu.sync_copy(rows_v, tab_s.at[idx_v], add=True)
                return carry
            return body

        @pl.when(cid == 0)
        def _():
            lax.fori_loop(0, nch, make_body(attrg_h), 0)

        @pl.when(cid == 1)
        def _():
            lax.fori_loop(0, nch, make_body(attrl_h), 0)

        plsc.subcore_barrier()

        def writeout(out_h):
            pltpu.sync_copy(tab_s.at[pl.ds(sid * rpb, rpb)],
                            out_h.at[pl.ds(sid * rpb, rpb)])
            if tail:
                @pl.when(sid == ns - 1)
                def _():
                    pltpu.sync_copy(tab_s.at[pl.ds(ns * rpb, tail)],
                                    out_h.at[pl.ds(ns * rpb, tail)])

        @pl.when(cid == 0)
        def _():
            writeout(outg_h)

        @pl.when(cid == 1)
        def _():
            writeout(outl_h)

    return scat(attr_g, attr_l, col)


def _nt_body(ng_ref, nl_ref, wgt_ref, wgbo_ref, bg1_ref,
             wlt_ref, wlbo_ref, bl1_ref, batch_ref, ts_ref, al_ref,
             gt_ref, gb_ref, lt_ref, lb_ref, cnt_ref, sa_ref):
    ng = ng_ref[...]
    nl = nl_ref[...]
    gt_ref[...] = jnp.dot(ng, wgt_ref[...], preferred_element_type=F32) + bg1_ref[...]
    gb_ref[...] = jnp.dot(ng, wgbo_ref[...], preferred_element_type=F32)
    lt_ref[...] = jnp.dot(nl, wlt_ref[...], preferred_element_type=F32) + bl1_ref[...]
    lb_ref[...] = jnp.dot(nl, wlbo_ref[...], preferred_element_type=F32)

    @pl.when(pl.program_id(0) == 0)
    def _():
        batch = batch_ref[...]
        n = batch.shape[0]
        b = cnt_ref.shape[1]
        lt = (batch < lax.broadcasted_iota(jnp.int32, (n, b), 1)).astype(F32)
        cnt_ref[...] = jnp.sum(lt, axis=0, keepdims=True)
        ts = ts_ref[...]
        t = al_ref.shape[0]
        oh = (ts == lax.broadcasted_iota(jnp.int32, (b, t), 1)).astype(F32)
        a = jnp.dot(oh, al_ref[...], preferred_element_type=F32)
        sa_ref[...] = jnp.sqrt(a) / jnp.sqrt(1.0 - a)


def _node_transform(node_g, node_l, wgt, wgb, bg1, wlt, wlb, bl1,
                    batch2, ts2, al2, nb=2000):
    N, H = node_g.shape
    B = ts2.shape[0]
    T = al2.shape[0]
    full = lambda s: pl.BlockSpec(s, lambda i: (0, 0))
    return pl.pallas_call(
        _nt_body,
        grid=(N // nb,),
        in_specs=[
            pl.BlockSpec((nb, H), lambda i: (i, 0)),
            pl.BlockSpec((nb, H), lambda i: (i, 0)),
            full((H, H)), full((H, H)), full((1, H)),
            full((H, H)), full((H, H)), full((1, H)),
            full((N, 1)), full((B, 1)), full((T, 1)),
        ],
        out_specs=[pl.BlockSpec((nb, H), lambda i: (i, 0))] * 4 + [
            full((1, B)), full((B, 1))],
        out_shape=[jax.ShapeDtypeStruct((N, H), F32)] * 4 + [
            jax.ShapeDtypeStruct((1, B), F32),
            jax.ShapeDtypeStruct((B, 1), F32)],
    )(node_g, node_l, wgt, wgb, bg1, wlt, wlb, bl1, batch2, ts2, al2)


def _sc_gather(gt, gb, ltab, lbot, row, col):
    N, H = gt.shape
    E = row.shape[0]
    info = plsc.get_sparse_core_info()
    ns = info.num_subcores
    ept = E // ns
    ch = 80
    nch = ept // ch
    mesh = plsc.VectorSubcoreMesh(core_axis_name="c", subcore_axis_name="s")

    @functools.partial(
        pl.kernel,
        out_type=(jax.ShapeDtypeStruct((E, H), F32),
                  jax.ShapeDtypeStruct((E, H), F32)),
        mesh=mesh,
        scratch_types=[
            pltpu.VMEM((ch,), jnp.int32),
            pltpu.VMEM((ch,), jnp.int32),
            pltpu.VMEM((ch, H), F32),
            pltpu.VMEM((ch, H), F32),
            pltpu.SemaphoreType.DMA,
            pltpu.SemaphoreType.DMA,
        ],
    )
    def gat(gt_h, gb_h, lt_h, lb_h, row_h, col_h, outg_h, outl_h,
            ridx, cidx, av, bv, s1, s2):
        cid = lax.axis_index("c")
        sid = lax.axis_index("s")

        def make_body(t1, t2, out_h):
            def body(i, carry):
                off = sid * ept + i * ch
                pltpu.sync_copy(row_h.at[pl.ds(off, ch)], ridx)
                pltpu.sync_copy(col_h.at[pl.ds(off, ch)], cidx)
                ca = pltpu.async_copy(t1.at[ridx], av, s1)
                cb = pltpu.async_copy(t2.at[cidx], bv, s2)
                ca.wait()
                cb.wait()

                def add_row(r, c2):
                    for j in range(H // 16):
                        sl = pl.ds(j * 16, 16)
                        av[r, sl] = av[r, sl] + bv[r, sl]
                    return c2
                lax.fori_loop(0, ch, add_row, 0)
                pltpu.sync_copy(av, out_h.at[pl.ds(off, ch)])
                return carry
            return body

        @pl.when(cid == 0)
        def _():
            lax.fori_loop(0, nch, make_body(gt_h, gb_h, outg_h), 0)

        @pl.when(cid == 1)
        def _():
            lax.fori_loop(0, nch, make_body(lt_h, lb_h, outl_h), 0)

    return gat(gt, gb, ltab, lbot, row, col)


def _mlp_body(h1g_ref, h1l_ref, row_ref, et_ref, cnt_ref, sa_ref,
              wg2_ref, bg2_ref, wg3_ref, bg3_ref,
              wl2_ref, bl2_ref, wl3_ref, bl3_ref, out_ref):
    def mlp(x, w2, b2, w3, b3):
        h = jnp.maximum(jnp.dot(jnp.maximum(x, 0.0), w2,
                                preferred_element_type=F32) + b2, 0.0)
        return jnp.dot(h, w3, preferred_element_type=F32) + b3

    og = mlp(h1g_ref[...], wg2_ref[...], bg2_ref[...], wg3_ref[...],
             bg3_ref[...])
    ol = mlp(h1l_ref[...], wl2_ref[...], bl2_ref[...], wl3_ref[...],
             bl3_ref[...])
    row = row_ref[...]
    eb = row.shape[0]
    b = cnt_ref.shape[1]
    cnt = cnt_ref[...].astype(jnp.int32)
    ge = (row >= cnt).astype(jnp.int32)
    e2g = jnp.sum(ge, axis=1, keepdims=True) - 1
    oh = (e2g == lax.broadcasted_iota(jnp.int32, (eb, b), 1)).astype(F32)
    scale = jnp.dot(oh, sa_ref[...], preferred_element_type=F32)
    mask = (et_ref[...] > 0).astype(F32)
    out_ref[...] = jnp.concatenate([og * scale, ol * mask], axis=1)


def _edge_mlp(h1g, h1l, row2, et2, cnt, sa,
              Wg2, bg2, Wg3, bg3, Wl2, bl2, Wl3, bl3, eb=2000):
    E, H = h1g.shape
    B = cnt.shape[1]
    K = Wg2.shape[1]
    full = lambda s: pl.BlockSpec(s, lambda i: (0, 0))
    return pl.pallas_call(
        _mlp_body,
        grid=(E // eb,),
        in_specs=[
            pl.BlockSpec((eb, H), lambda i: (i, 0)),
            pl.BlockSpec((eb, H), lambda i: (i, 0)),
            pl.BlockSpec((eb, 1), lambda i: (i, 0)),
            pl.BlockSpec((eb, 1), lambda i: (i, 0)),
            full((1, B)), full((B, 1)),
            full((H, K)), full((1, K)), full((K, 1)), full((1, 1)),
            full((H, K)), full((1, K)), full((K, 1)), full((1, 1)),
        ],
        out_specs=pl.BlockSpec((eb, 2), lambda i: (i, 0)),
        out_shape=jax.ShapeDtypeStruct((E, 2), F32),
    )(h1g, h1l, row2, et2, cnt, sa,
      Wg2, bg2, Wg3, bg3, Wl2, bl2, Wl3, bl3)


def kernel(atom_type, pos, bond_index, bond_type, batch, time_step,
           edge_index, edge_type, edge_length, alphas,
           emb_g, Wg_a, bg_a, Wg_b, bg_b, emb_l, Wl_a, bl_a, Wl_b, bl_b,
           Wg1, bg1, Wg2, bg2, Wg3, bg3, Wl1, bl1, Wl2, bl2, Wl3, bl3):
    E = edge_index.shape[1]
    N = batch.shape[0]
    H = emb_g.shape[1]
    row = edge_index[0].astype(jnp.int32)
    col = edge_index[1].astype(jnp.int32)
    et2 = edge_type.astype(jnp.int32).reshape(E, 1)
    row2 = row.reshape(E, 1)
    el = edge_length.astype(F32)

    attr_g, attr_l = _edge_encode(
        el, et2, Wg_a, bg_a.reshape(1, H), Wg_b, bg_b.reshape(1, H), emb_g,
        Wl_a, bl_a.reshape(1, H), Wl_b, bl_b.reshape(1, H), emb_l)

    node_g, node_l = _sc_scatter(attr_g, attr_l, col, N)

    gt, gb, ltab, lbot, cnt, sa = _node_transform(
        node_g, node_l, Wg1[:H], Wg1[H:], bg1.reshape(1, H),
        Wl1[:H], Wl1[H:], bl1.reshape(1, H),
        batch.astype(jnp.int32).reshape(N, 1),
        time_step.astype(jnp.int32).reshape(-1, 1),
        alphas.astype(F32).reshape(-1, 1))

    h1g, h1l = _sc_gather(gt, gb, ltab, lbot, row, col)

    return _edge_mlp(h1g, h1l, row2, et2, cnt, sa,
                     Wg2, bg2.reshape(1, -1), Wg3, bg3.reshape(1, 1),
                     Wl2, bl2.reshape(1, -1), Wl3, bl3.reshape(1, 1))


# speedup vs baseline: 3.9713x; 1.4175x over previous
import functools

import jax
import jax.numpy as jnp
from jax import lax
from jax.experimental import pallas as pl
from jax.experimental.pallas import tpu as pltpu
from jax.experimental.pallas import tpu_sc as plsc

F32 = jnp.float32


def _enc_body(el_ref, et_ref, wga_ref, bga_ref, wgb_ref, bgb_ref, embg_ref,
              wla_ref, bla_ref, wlb_ref, blb_ref, embl_ref,
              outg_ref, outl_ref):
    el = el_ref[...]
    et = et_ref[...]
    eb = el.shape[0]
    ncls = embg_ref.shape[0]
    oh = (et == lax.broadcasted_iota(jnp.int32, (eb, ncls), 1)).astype(F32)

    def enc(wa, ba, wb, bb, emb):
        h = jnp.maximum(el * wa + ba, 0.0)
        d = jnp.dot(h, wb, preferred_element_type=F32) + bb
        return d * jnp.dot(oh, emb, preferred_element_type=F32)

    outg_ref[...] = enc(wga_ref[...], bga_ref[...], wgb_ref[...],
                        bgb_ref[...], embg_ref[...])
    outl_ref[...] = enc(wla_ref[...], bla_ref[...], wlb_ref[...],
                        blb_ref[...], embl_ref[...])


def _edge_encode(el, et2, Wg_a, bg_a, Wg_b, bg_b, emb_g,
                 Wl_a, bl_a, Wl_b, bl_b, emb_l, eb=2000):
    E = el.shape[0]
    H = emb_g.shape[1]
    C = emb_g.shape[0]
    full = lambda s: pl.BlockSpec(s, lambda i: (0, 0))
    return pl.pallas_call(
        _enc_body,
        grid=(E // eb,),
        in_specs=[
            pl.BlockSpec((eb, 1), lambda i: (i, 0)),
            pl.BlockSpec((eb, 1), lambda i: (i, 0)),
            full((1, H)), full((1, H)), full((H, H)), full((1, H)),
            full((C, H)),
            full((1, H)), full((1, H)), full((H, H)), full((1, H)),
            full((C, H)),
        ],
        out_specs=[pl.BlockSpec((eb, H), lambda i: (i, 0))] * 2,
        out_shape=[jax.ShapeDtypeStruct((E, H), F32)] * 2,
    )(el, et2, Wg_a, bg_a, Wg_b, bg_b, emb_g,
      Wl_a, bl_a, Wl_b, bl_b, emb_l)


def _sc_scatter(attr_g, attr_l, col, N):
    E, H = attr_g.shape
    ch = 80
    info = plsc.get_sparse_core_info()
    ns = info.num_subcores
    ept = E // ns
    nch = ept // ch
    npair = nch // 2
    rpb = (N // (8 * ns)) * 8
    tail = N - rpb * ns
    nz = rpb // ch
    zrem = rpb - nz * ch
    mesh = plsc.VectorSubcoreMesh(core_axis_name="c", subcore_axis_name="s")

    @functools.partial(
        pl.kernel,
        out_type=(jax.ShapeDtypeStruct((N, H), F32),
                  jax.ShapeDtypeStruct((N, H), F32)),
        mesh=mesh,
        scratch_types=[
            pltpu.VMEM((ch,), jnp.int32),
            pltpu.VMEM((ch,), jnp.int32),
            pltpu.VMEM((2, ch, H), F32),
            pltpu.VMEM_SHARED((N, H), F32),
            pltpu.SemaphoreType.DMA,
            pltpu.SemaphoreType.DMA,
            pltpu.SemaphoreType.DMA,
        ],
    )
    def scat(attrg_h, attrl_h, col_h, outg_h, outl_h,
             idx0, idx1, rows_v, tab_s, rsem, isem, ssem):
        cid = lax.axis_index("c")
        sid = lax.axis_index("s")

        def zrow(r, carry):
            for j in range(H // 16):
                rows_v[0, r, pl.ds(j * 16, 16)] = jnp.zeros((16,), F32)
            return carry
        lax.fori_loop(0, ch, zrow, 0)

        def zcp(k, carry):
            pltpu.sync_copy(rows_v.at[0],
                            tab_s.at[pl.ds(sid * rpb + k * ch, ch)])
            return carry
        lax.fori_loop(0, nz, zcp, 0)
        if zrem:
            pltpu.sync_copy(rows_v.at[0, pl.ds(0, zrem)],
                            tab_s.at[pl.ds(sid * rpb + nz * ch, zrem)])
        if tail:
            @pl.when(sid == ns - 1)
            def _():
                pltpu.sync_copy(rows_v.at[0, pl.ds(0, tail)],
                                tab_s.at[pl.ds(ns * rpb, tail)])
        plsc.subcore_barrier()

        def run(attr_h):
            base = sid * ept
            pltpu.async_copy(col_h.at[pl.ds(base, ch)], idx0, isem)
            pltpu.async_copy(attr_h.at[pl.ds(base, ch)], rows_v.at[0], rsem)

            def wait_rows(b):
                pltpu.make_async_copy(attr_h.at[pl.ds(0, ch)],
                                      rows_v.at[b], rsem).wait()

            def wait_idx(b):
                pltpu.make_async_copy(col_h.at[pl.ds(0, ch)],
                                      idx0 if b == 0 else idx1, isem).wait()

            def drain_scat():
                pltpu.make_async_copy(attr_h.at[pl.ds(0, ch)],
                                      rows_v.at[0], ssem).wait()

            def pair(i, carry):
                for b in range(2):
                    j = 2 * i + b
                    off = base + j * ch

                    @pl.when(j >= 1)
                    def _():
                        drain_scat()

                    @pl.when(j + 1 < nch)
                    def _():
                        pltpu.async_copy(col_h.at[pl.ds(off + ch, ch)],
                                         idx1 if b == 0 else idx0, isem)
                        pltpu.async_copy(attr_h.at[pl.ds(off + ch, ch)],
                                         rows_v.at[1 - b], rsem)
                    wait_idx(b)
                    wait_rows(b)
                    pltpu.async_copy(rows_v.at[b],
                                     tab_s.at[idx0 if b == 0 else idx1],
                                     ssem, add=True)
                return carry
            lax.fori_loop(0, npair, pair, 0)
            drain_scat()

        @pl.when(cid == 0)
        def _():
            run(attrg_h)

        @pl.when(cid == 1)
        def _():
            run(attrl_h)

        plsc.subcore_barrier()

        def writeout(out_h):
            pltpu.sync_copy(tab_s.at[pl.ds(sid * rpb, rpb)],
                            out_h.at[pl.ds(sid * rpb, rpb)])
            if tail:
                @pl.when(sid == ns - 1)
                def _():
                    pltpu.sync_copy(tab_s.at[pl.ds(ns * rpb, tail)],
                                    out_h.at[pl.ds(ns * rpb, tail)])

        @pl.when(cid == 0)
        def _():
            writeout(outg_h)

        @pl.when(cid == 1)
        def _():
            writeout(outl_h)

    return scat(attr_g, attr_l, col)


def _nt_body(ng_ref, nl_ref, wgt_ref, wgbo_ref, bg1_ref,
             wlt_ref, wlbo_ref, bl1_ref, batch_ref, ts_ref, al_ref,
             gt_ref, gb_ref, lt_ref, lb_ref, cnt_ref, sa_ref):
    ng = ng_ref[...]
    nl = nl_ref[...]
    gt_ref[...] = jnp.dot(ng, wgt_ref[...], preferred_element_type=F32) + bg1_ref[...]
    gb_ref[...] = jnp.dot(ng, wgbo_ref[...], preferred_element_type=F32)
    lt_ref[...] = jnp.dot(nl, wlt_ref[...], preferred_element_type=F32) + bl1_ref[...]
    lb_ref[...] = jnp.dot(nl, wlbo_ref[...], preferred_element_type=F32)

    @pl.when(pl.program_id(0) == 0)
    def _():
        batch = batch_ref[...]
        n = batch.shape[0]
        b = cnt_ref.shape[1]
        lt = (batch < lax.broadcasted_iota(jnp.int32, (n, b), 1)).astype(F32)
        cnt_ref[...] = jnp.sum(lt, axis=0, keepdims=True)
        ts = ts_ref[...]
        t = al_ref.shape[0]
        oh = (ts == lax.broadcasted_iota(jnp.int32, (b, t), 1)).astype(F32)
        a = jnp.dot(oh, al_ref[...], preferred_element_type=F32)
        sa_ref[...] = jnp.sqrt(a) / jnp.sqrt(1.0 - a)


def _node_transform(node_g, node_l, wgt, wgb, bg1, wlt, wlb, bl1,
                    batch2, ts2, al2, nb=2000):
    N, H = node_g.shape
    B = ts2.shape[0]
    T = al2.shape[0]
    full = lambda s: pl.BlockSpec(s, lambda i: (0, 0))
    return pl.pallas_call(
        _nt_body,
        grid=(N // nb,),
        in_specs=[
            pl.BlockSpec((nb, H), lambda i: (i, 0)),
            pl.BlockSpec((nb, H), lambda i: (i, 0)),
            full((H, H)), full((H, H)), full((1, H)),
            full((H, H)), full((H, H)), full((1, H)),
            full((N, 1)), full((B, 1)), full((T, 1)),
        ],
        out_specs=[pl.BlockSpec((nb, H), lambda i: (i, 0))] * 4 + [
            full((1, B)), full((B, 1))],
        out_shape=[jax.ShapeDtypeStruct((N, H), F32)] * 4 + [
            jax.ShapeDtypeStruct((1, B), F32),
            jax.ShapeDtypeStruct((B, 1), F32)],
    )(node_g, node_l, wgt, wgb, bg1, wlt, wlb, bl1, batch2, ts2, al2)


def _sc_gather(gt, gb, ltab, lbot, row3, col3, E):
    N, H = gt.shape
    ch = row3.shape[2]
    nch = row3.shape[1]
    info = plsc.get_sparse_core_info()
    ns = info.num_subcores
    ept = E // ns
    npair = nch // 2
    mesh = plsc.VectorSubcoreMesh(core_axis_name="c", subcore_axis_name="s")

    @functools.partial(
        pl.kernel,
        out_type=(jax.ShapeDtypeStruct((E, H), F32),
                  jax.ShapeDtypeStruct((E, H), F32)),
        mesh=mesh,
        scratch_types=[
            pltpu.VMEM((nch, ch), jnp.int32),
            pltpu.VMEM((nch, ch), jnp.int32),
            pltpu.VMEM((2, ch, H), F32),
            pltpu.VMEM((2, ch, H), F32),
            pltpu.SemaphoreType.DMA,
            pltpu.SemaphoreType.DMA,
        ],
    )
    def gat(gt_h, gb_h, lt_h, lb_h, row_h, col_h, outg_h, outl_h,
            rowv, colv, av, bv, gsem, wsem):
        cid = lax.axis_index("c")
        sid = lax.axis_index("s")

        def run(t1, t2, out_h):
            pltpu.sync_copy(row_h.at[sid], rowv)
            pltpu.sync_copy(col_h.at[sid], colv)
            pltpu.async_copy(t1.at[rowv.at[0]], av.at[0], gsem)
            pltpu.async_copy(t2.at[colv.at[0]], bv.at[0], gsem)

            def wait_g(b):
                pltpu.make_async_copy(out_h.at[pl.ds(0, ch)],
                                      av.at[b], gsem).wait()
                pltpu.make_async_copy(out_h.at[pl.ds(0, ch)],
                                      bv.at[b], gsem).wait()

            def drain_w():
                pltpu.make_async_copy(out_h.at[pl.ds(0, ch)],
                                      av.at[0], wsem).wait()

            def pair(i, carry):
                for b in range(2):
                    j = 2 * i + b
                    off = sid * ept + j * ch
                    wait_g(b)

                    @pl.when(j + 1 < nch)
                    def _():
                        @pl.when(j >= 1)
                        def _():
                            drain_w()
                        pltpu.async_copy(t1.at[rowv.at[j + 1]],
                                         av.at[1 - b], gsem)
                        pltpu.async_copy(t2.at[colv.at[j + 1]],
                                         bv.at[1 - b], gsem)

                    ab = av.at[b]
                    bb = bv.at[b]

                    def add2(r, c2):
                        for rr in range(2):
                            for jj in range(H // 16):
                                sl = pl.ds(jj * 16, 16)
                                ab[2 * r + rr, sl] = (ab[2 * r + rr, sl]
                                                      + bb[2 * r + rr, sl])
                        return c2
                    lax.fori_loop(0, ch // 2, add2, 0)
                    pltpu.async_copy(ab, out_h.at[pl.ds(off, ch)], wsem)
                return carry
            lax.fori_loop(0, npair, pair, 0)
            drain_w()
            drain_w()

        @pl.when(cid == 0)
        def _():
            run(gt_h, gb_h, outg_h)

        @pl.when(cid == 1)
        def _():
            run(lt_h, lb_h, outl_h)

    return gat(gt, gb, ltab, lbot, row3, col3)


def _mlp_body(h1g_ref, h1l_ref, row_ref, et_ref, cnt_ref, sa_ref,
              wg2_ref, bg2_ref, wg3_ref, bg3_ref,
              wl2_ref, bl2_ref, wl3_ref, bl3_ref, out_ref):
    def mlp(x, w2, b2, w3, b3):
        h = jnp.maximum(jnp.dot(jnp.maximum(x, 0.0), w2,
                                preferred_element_type=F32) + b2, 0.0)
        return jnp.dot(h, w3, preferred_element_type=F32) + b3

    og = mlp(h1g_ref[...], wg2_ref[...], bg2_ref[...], wg3_ref[...],
             bg3_ref[...])
    ol = mlp(h1l_ref[...], wl2_ref[...], bl2_ref[...], wl3_ref[...],
             bl3_ref[...])
    row = row_ref[...]
    eb = row.shape[0]
    b = cnt_ref.shape[1]
    cnt = cnt_ref[...].astype(jnp.int32)
    ge = (row >= cnt).astype(jnp.int32)
    e2g = jnp.sum(ge, axis=1, keepdims=True) - 1
    oh = (e2g == lax.broadcasted_iota(jnp.int32, (eb, b), 1)).astype(F32)
    scale = jnp.dot(oh, sa_ref[...], preferred_element_type=F32)
    mask = (et_ref[...] > 0).astype(F32)
    out_ref[...] = jnp.concatenate([og * scale, ol * mask], axis=1)


def _edge_mlp(h1g, h1l, row2, et2, cnt, sa,
              Wg2, bg2, Wg3, bg3, Wl2, bl2, Wl3, bl3, eb=2000):
    E, H = h1g.shape
    B = cnt.shape[1]
    K = Wg2.shape[1]
    full = lambda s: pl.BlockSpec(s, lambda i: (0, 0))
    return pl.pallas_call(
        _mlp_body,
        grid=(E // eb,),
        in_specs=[
            pl.BlockSpec((eb, H), lambda i: (i, 0)),
            pl.BlockSpec((eb, H), lambda i: (i, 0)),
            pl.BlockSpec((eb, 1), lambda i: (i, 0)),
            pl.BlockSpec((eb, 1), lambda i: (i, 0)),
            full((1, B)), full((B, 1)),
            full((H, K)), full((1, K)), full((K, 1)), full((1, 1)),
            full((H, K)), full((1, K)), full((K, 1)), full((1, 1)),
        ],
        out_specs=pl.BlockSpec((eb, 2), lambda i: (i, 0)),
        out_shape=jax.ShapeDtypeStruct((E, 2), F32),
    )(h1g, h1l, row2, et2, cnt, sa,
      Wg2, bg2, Wg3, bg3, Wl2, bl2, Wl3, bl3)


def kernel(atom_type, pos, bond_index, bond_type, batch, time_step,
           edge_index, edge_type, edge_length, alphas,
           emb_g, Wg_a, bg_a, Wg_b, bg_b, emb_l, Wl_a, bl_a, Wl_b, bl_b,
           Wg1, bg1, Wg2, bg2, Wg3, bg3, Wl1, bl1, Wl2, bl2, Wl3, bl3):
    E = edge_index.shape[1]
    N = batch.shape[0]
    H = emb_g.shape[1]
    ch = 80
    row = edge_index[0].astype(jnp.int32)
    col = edge_index[1].astype(jnp.int32)
    et2 = edge_type.astype(jnp.int32).reshape(E, 1)
    row2 = row.reshape(E, 1)
    rowc = row.reshape(16, E // (16 * ch), ch)
    colc = col.reshape(16, E // (16 * ch), ch)
    el = edge_length.astype(F32)

    attr_g, attr_l = _edge_encode(
        el, et2, Wg_a, bg_a.reshape(1, H), Wg_b, bg_b.reshape(1, H), emb_g,
        Wl_a, bl_a.reshape(1, H), Wl_b, bl_b.reshape(1, H), emb_l)

    node_g, node_l = _sc_scatter(attr_g, attr_l, col, N)

    gt, gb, ltab, lbot, cnt, sa = _node_transform(
        node_g, node_l, Wg1[:H], Wg1[H:], bg1.reshape(1, H),
        Wl1[:H], Wl1[H:], bl1.reshape(1, H),
        batch.astype(jnp.int32).reshape(N, 1),
        time_step.astype(jnp.int32).reshape(-1, 1),
        alphas.astype(F32).reshape(-1, 1))

    h1g, h1l = _sc_gather(gt, gb, ltab, lbot, rowc, colc, E)

    return _edge_mlp(h1g, h1l, row2, et2, cnt, sa,
                     Wg2, bg2.reshape(1, -1), Wg3, bg3.reshape(1, 1),
                     Wl2, bl2.reshape(1, -1), Wl3, bl3.reshape(1, 1))


# speedup vs baseline: 4.3475x; 1.0947x over previous
import functools

import jax
import jax.numpy as jnp
from jax import lax
from jax.experimental import pallas as pl
from jax.experimental.pallas import tpu as pltpu
from jax.experimental.pallas import tpu_sc as plsc

F32 = jnp.float32


def _lane_onehots(k, eb):
    u = (lax.broadcasted_iota(jnp.int32, (eb, k), 0) // 128 ==
         lax.broadcasted_iota(jnp.int32, (eb, k), 1)).astype(F32)
    v = (lax.broadcasted_iota(jnp.int32, (eb, 128), 0) % 128 ==
         lax.broadcasted_iota(jnp.int32, (eb, 128), 1)).astype(F32)
    return u, v


def _expand_col(xw, u, v):
    y2 = jnp.dot(u, xw, preferred_element_type=F32)
    return jnp.sum(y2 * v, axis=1, keepdims=True)


def _pack_lanes(ycol, u, v, k):
    return jnp.dot(u.T, ycol * v, preferred_element_type=F32)


def _enc_body(el_ref, et_ref, wga_ref, bga_ref, wgb_ref, bgb_ref, embg_ref,
              wla_ref, bla_ref, wlb_ref, blb_ref, embl_ref,
              outg_ref, outl_ref):
    elw = el_ref[0]
    etw = et_ref[0].astype(F32)
    k = elw.shape[0]
    eb = k * 128
    ncls = embg_ref.shape[0]
    u, v = _lane_onehots(k, eb)
    el = _expand_col(elw, u, v)
    et = _expand_col(etw, u, v)
    ioc = lax.broadcasted_iota(jnp.int32, (eb, ncls), 1).astype(F32)
    oh = (et == ioc).astype(F32)

    def enc(wa, ba, wb, bb, emb):
        h = jnp.maximum(el * wa + ba, 0.0)
        d = jnp.dot(h, wb, preferred_element_type=F32) + bb
        return d * jnp.dot(oh, emb, preferred_element_type=F32)

    outg_ref[...] = enc(wga_ref[...], bga_ref[...], wgb_ref[...],
                        bgb_ref[...], embg_ref[...])
    outl_ref[...] = enc(wla_ref[...], bla_ref[...], wlb_ref[...],
                        blb_ref[...], embl_ref[...])


def _edge_encode(el_l, et_l, Wg_a, bg_a, Wg_b, bg_b, emb_g,
                 Wl_a, bl_a, Wl_b, bl_b, emb_l):
    G, k, L = el_l.shape
    E = G * k * L
    eb = k * L
    H = emb_g.shape[1]
    C = emb_g.shape[0]
    full = lambda s: pl.BlockSpec(s, lambda i: (0, 0))
    return pl.pallas_call(
        _enc_body,
        grid=(G,),
        in_specs=[
            pl.BlockSpec((1, k, L), lambda i: (i, 0, 0)),
            pl.BlockSpec((1, k, L), lambda i: (i, 0, 0)),
            full((1, H)), full((1, H)), full((H, H)), full((1, H)),
            full((C, H)),
            full((1, H)), full((1, H)), full((H, H)), full((1, H)),
            full((C, H)),
        ],
        out_specs=[pl.BlockSpec((eb, H), lambda i: (i, 0))] * 2,
        out_shape=[jax.ShapeDtypeStruct((E, H), F32)] * 2,
    )(el_l, et_l, Wg_a, bg_a, Wg_b, bg_b, emb_g,
      Wl_a, bl_a, Wl_b, bl_b, emb_l)


def _sc_scatter(attr_g, attr_l, col, N):
    E, H = attr_g.shape
    ch = 80
    info = plsc.get_sparse_core_info()
    ns = info.num_subcores
    ept = E // ns
    nch = ept // ch
    npair = nch // 2
    rpb = (N // (8 * ns)) * 8
    tail = N - rpb * ns
    nz = rpb // ch
    zrem = rpb - nz * ch
    mesh = plsc.VectorSubcoreMesh(core_axis_name="c", subcore_axis_name="s")

    @functools.partial(
        pl.kernel,
        out_type=(jax.ShapeDtypeStruct((N, H), F32),
                  jax.ShapeDtypeStruct((N, H), F32)),
        mesh=mesh,
        scratch_types=[
            pltpu.VMEM((ch,), jnp.int32),
            pltpu.VMEM((ch,), jnp.int32),
            pltpu.VMEM((2, ch, H), F32),
            pltpu.VMEM_SHARED((N, H), F32),
            pltpu.SemaphoreType.DMA,
            pltpu.SemaphoreType.DMA,
            pltpu.SemaphoreType.DMA,
        ],
    )
    def scat(attrg_h, attrl_h, col_h, outg_h, outl_h,
             idx0, idx1, rows_v, tab_s, rsem, isem, ssem):
        cid = lax.axis_index("c")
        sid = lax.axis_index("s")

        def zrow(r, carry):
            for j in range(H // 16):
                rows_v[0, r, pl.ds(j * 16, 16)] = jnp.zeros((16,), F32)
            return carry
        lax.fori_loop(0, ch, zrow, 0)

        def zcp(k, carry):
            pltpu.sync_copy(rows_v.at[0],
                            tab_s.at[pl.ds(sid * rpb + k * ch, ch)])
            return carry
        lax.fori_loop(0, nz, zcp, 0)
        if zrem:
            pltpu.sync_copy(rows_v.at[0, pl.ds(0, zrem)],
                            tab_s.at[pl.ds(sid * rpb + nz * ch, zrem)])
        if tail:
            @pl.when(sid == ns - 1)
            def _():
                pltpu.sync_copy(rows_v.at[0, pl.ds(0, tail)],
                                tab_s.at[pl.ds(ns * rpb, tail)])
        plsc.subcore_barrier()

        def run(attr_h):
            base = sid * ept
            pltpu.async_copy(col_h.at[pl.ds(base, ch)], idx0, isem)
            pltpu.async_copy(attr_h.at[pl.ds(base, ch)], rows_v.at[0], rsem)

            def wait_rows(b):
                pltpu.make_async_copy(attr_h.at[pl.ds(0, ch)],
                                      rows_v.at[b], rsem).wait()

            def wait_idx(b):
                pltpu.make_async_copy(col_h.at[pl.ds(0, ch)],
                                      idx0 if b == 0 else idx1, isem).wait()

            def drain_scat():
                pltpu.make_async_copy(attr_h.at[pl.ds(0, ch)],
                                      rows_v.at[0], ssem).wait()

            def pair(i, carry):
                for b in range(2):
                    j = 2 * i + b
                    off = base + j * ch

                    @pl.when(j >= 1)
                    def _():
                        drain_scat()

                    @pl.when(j + 1 < nch)
                    def _():
                        pltpu.async_copy(col_h.at[pl.ds(off + ch, ch)],
                                         idx1 if b == 0 else idx0, isem)
                        pltpu.async_copy(attr_h.at[pl.ds(off + ch, ch)],
                                         rows_v.at[1 - b], rsem)
                    wait_idx(b)
                    wait_rows(b)
                    pltpu.async_copy(rows_v.at[b],
                                     tab_s.at[idx0 if b == 0 else idx1],
                                     ssem, add=True)
                return carry
            lax.fori_loop(0, npair, pair, 0)
            drain_scat()

        @pl.when(cid == 0)
        def _():
            run(attrg_h)

        @pl.when(cid == 1)
        def _():
            run(attrl_h)

        plsc.subcore_barrier()

        def writeout(out_h):
            pltpu.sync_copy(tab_s.at[pl.ds(sid * rpb, rpb)],
                            out_h.at[pl.ds(sid * rpb, rpb)])
            if tail:
                @pl.when(sid == ns - 1)
                def _():
                    pltpu.sync_copy(tab_s.at[pl.ds(ns * rpb, tail)],
                                    out_h.at[pl.ds(ns * rpb, tail)])

        @pl.when(cid == 0)
        def _():
            writeout(outg_h)

        @pl.when(cid == 1)
        def _():
            writeout(outl_h)

    return scat(attr_g, attr_l, col)


def _nt_body(ng_ref, nl_ref, wgt_ref, wgbo_ref, bg1_ref,
             wlt_ref, wlbo_ref, bl1_ref, batch_ref, ts_ref, al_ref,
             gt_ref, gb_ref, lt_ref, lb_ref, cnt_ref, sa_ref):
    ng = ng_ref[...]
    nl = nl_ref[...]
    gt_ref[...] = jnp.dot(ng, wgt_ref[...], preferred_element_type=F32) + bg1_ref[...]
    gb_ref[...] = jnp.dot(ng, wgbo_ref[...], preferred_element_type=F32)
    lt_ref[...] = jnp.dot(nl, wlt_ref[...], preferred_element_type=F32) + bl1_ref[...]
    lb_ref[...] = jnp.dot(nl, wlbo_ref[...], preferred_element_type=F32)

    @pl.when(pl.program_id(0) == 0)
    def _():
        batch = batch_ref[...]
        n = batch.shape[0]
        b = cnt_ref.shape[1]
        lt = (batch < lax.broadcasted_iota(jnp.int32, (n, b), 1)).astype(F32)
        cnt_ref[...] = jnp.sum(lt, axis=0, keepdims=True)
        ts = ts_ref[...]
        t = al_ref.shape[0]
        oh = (ts == lax.broadcasted_iota(jnp.int32, (b, t), 1)).astype(F32)
        a = jnp.dot(oh, al_ref[...], preferred_element_type=F32)
        sa_ref[...] = jnp.sqrt(a) / jnp.sqrt(1.0 - a)


def _node_transform(node_g, node_l, wgt, wgb, bg1, wlt, wlb, bl1,
                    batch2, ts2, al2, nb=2000):
    N, H = node_g.shape
    B = ts2.shape[0]
    T = al2.shape[0]
    full = lambda s: pl.BlockSpec(s, lambda i: (0, 0))
    return pl.pallas_call(
        _nt_body,
        grid=(N // nb,),
        in_specs=[
            pl.BlockSpec((nb, H), lambda i: (i, 0)),
            pl.BlockSpec((nb, H), lambda i: (i, 0)),
            full((H, H)), full((H, H)), full((1, H)),
            full((H, H)), full((H, H)), full((1, H)),
            full((N, 1)), full((B, 1)), full((T, 1)),
        ],
        out_specs=[pl.BlockSpec((nb, H), lambda i: (i, 0))] * 4 + [
            full((1, B)), full((B, 1))],
        out_shape=[jax.ShapeDtypeStruct((N, H), F32)] * 4 + [
            jax.ShapeDtypeStruct((1, B), F32),
            jax.ShapeDtypeStruct((B, 1), F32)],
    )(node_g, node_l, wgt, wgb, bg1, wlt, wlb, bl1, batch2, ts2, al2)


def _sc_gather(gt, gb, ltab, lbot, row3, col3, E):
    N, H = gt.shape
    ch = row3.shape[2]
    nch = row3.shape[1]
    info = plsc.get_sparse_core_info()
    ns = info.num_subcores
    ept = E // ns
    npair = nch // 2
    mesh = plsc.VectorSubcoreMesh(core_axis_name="c", subcore_axis_name="s")

    @functools.partial(
        pl.kernel,
        out_type=(jax.ShapeDtypeStruct((E, H), F32),
                  jax.ShapeDtypeStruct((E, H), F32)),
        mesh=mesh,
        scratch_types=[
            pltpu.VMEM((nch, ch), jnp.int32),
            pltpu.VMEM((nch, ch), jnp.int32),
            pltpu.VMEM((2, ch, H), F32),
            pltpu.VMEM((2, ch, H), F32),
            pltpu.SemaphoreType.DMA,
            pltpu.SemaphoreType.DMA,
        ],
    )
    def gat(gt_h, gb_h, lt_h, lb_h, row_h, col_h, outg_h, outl_h,
            rowv, colv, av, bv, gsem, wsem):
        cid = lax.axis_index("c")
        sid = lax.axis_index("s")

        def run(t1, t2, out_h):
            pltpu.sync_copy(row_h.at[sid], rowv)
            pltpu.sync_copy(col_h.at[sid], colv)
            pltpu.async_copy(t1.at[rowv.at[0]], av.at[0], gsem)
            pltpu.async_copy(t2.at[colv.at[0]], bv.at[0], gsem)

            def wait_g(b):
                pltpu.make_async_copy(out_h.at[pl.ds(0, ch)],
                                      av.at[b], gsem).wait()
                pltpu.make_async_copy(out_h.at[pl.ds(0, ch)],
                                      bv.at[b], gsem).wait()

            def drain_w():
                pltpu.make_async_copy(out_h.at[pl.ds(0, ch)],
                                      av.at[0], wsem).wait()

            def pair(i, carry):
                for b in range(2):
                    j = 2 * i + b
                    off = sid * ept + j * ch
                    wait_g(b)

                    @pl.when(j + 1 < nch)
                    def _():
                        @pl.when(j >= 1)
                        def _():
                            drain_w()
                        pltpu.async_copy(t1.at[rowv.at[j + 1]],
                                         av.at[1 - b], gsem)
                        pltpu.async_copy(t2.at[colv.at[j + 1]],
                                         bv.at[1 - b], gsem)

                    ab = av.at[b]
                    bb = bv.at[b]

                    def add2(r, c2):
                        for rr in range(2):
                            for jj in range(H // 16):
                                sl = pl.ds(jj * 16, 16)
                                ab[2 * r + rr, sl] = (ab[2 * r + rr, sl]
                                                      + bb[2 * r + rr, sl])
                        return c2
                    lax.fori_loop(0, ch // 2, add2, 0)
                    pltpu.async_copy(ab, out_h.at[pl.ds(off, ch)], wsem)
                return carry
            lax.fori_loop(0, npair, pair, 0)
            drain_w()
            drain_w()

        @pl.when(cid == 0)
        def _():
            run(gt_h, gb_h, outg_h)

        @pl.when(cid == 1)
        def _():
            run(lt_h, lb_h, outl_h)

    return gat(gt, gb, ltab, lbot, row3, col3)


def _mlp_body(h1g_ref, h1l_ref, roww_ref, etw_ref, cnt_ref, sa_ref,
              wg2_ref, bg2_ref, wg3_ref, bg3_ref,
              wl2_ref, bl2_ref, wl3_ref, bl3_ref, og_ref, ol_ref):
    def mlp(x, w2, b2, w3, b3):
        h = jnp.maximum(jnp.dot(jnp.maximum(x, 0.0), w2,
                                preferred_element_type=F32) + b2, 0.0)
        return jnp.dot(h, w3, preferred_element_type=F32) + b3

    og = mlp(h1g_ref[...], wg2_ref[...], bg2_ref[...], wg3_ref[...],
             bg3_ref[...])
    ol = mlp(h1l_ref[...], wl2_ref[...], bl2_ref[...], wl3_ref[...],
             bl3_ref[...])
    roww = roww_ref[0].astype(F32)
    etw = etw_ref[0].astype(F32)
    k = roww.shape[0]
    eb = k * 128
    b = cnt_ref.shape[1]
    u, v = _lane_onehots(k, eb)
    row = _expand_col(roww, u, v)
    et = _expand_col(etw, u, v)
    cnt = cnt_ref[...]
    ge = (row >= cnt).astype(F32)
    e2g = jnp.sum(ge, axis=1, keepdims=True) - 1.0
    iob = lax.broadcasted_iota(jnp.int32, (eb, b), 1).astype(F32)
    oh = (e2g == iob).astype(F32)
    scale = jnp.dot(oh, sa_ref[...], preferred_element_type=F32)
    mask = (et > 0.5).astype(F32)
    og_ref[0] = _pack_lanes(og * scale, u, v, k)
    ol_ref[0] = _pack_lanes(ol * mask, u, v, k)


def _edge_mlp(h1g, h1l, row_l, et_l, cnt, sa,
              Wg2, bg2, Wg3, bg3, Wl2, bl2, Wl3, bl3):
    E, H = h1g.shape
    G, k, L = row_l.shape
    eb = k * L
    B = cnt.shape[1]
    K = Wg2.shape[1]
    full = lambda s: pl.BlockSpec(s, lambda i: (0, 0))
    return pl.pallas_call(
        _mlp_body,
        grid=(G,),
        in_specs=[
            pl.BlockSpec((eb, H), lambda i: (i, 0)),
            pl.BlockSpec((eb, H), lambda i: (i, 0)),
            pl.BlockSpec((1, k, L), lambda i: (i, 0, 0)),
            pl.BlockSpec((1, k, L), lambda i: (i, 0, 0)),
            full((1, B)), full((B, 1)),
            full((H, K)), full((1, K)), full((K, 1)), full((1, 1)),
            full((H, K)), full((1, K)), full((K, 1)), full((1, 1)),
        ],
        out_specs=[pl.BlockSpec((1, k, L), lambda i: (i, 0, 0))] * 2,
        out_shape=[jax.ShapeDtypeStruct((G, k, L), F32)] * 2,
    )(h1g, h1l, row_l, et_l, cnt, sa,
      Wg2, bg2, Wg3, bg3, Wl2, bl2, Wl3, bl3)


def kernel(atom_type, pos, bond_index, bond_type, batch, time_step,
           edge_index, edge_type, edge_length, alphas,
           emb_g, Wg_a, bg_a, Wg_b, bg_b, emb_l, Wl_a, bl_a, Wl_b, bl_b,
           Wg1, bg1, Wg2, bg2, Wg3, bg3, Wl1, bl1, Wl2, bl2, Wl3, bl3):
    E = edge_index.shape[1]
    N = batch.shape[0]
    H = emb_g.shape[1]
    ch = 80
    L = 128
    row = edge_index[0].astype(jnp.int32)
    col = edge_index[1].astype(jnp.int32)
    kk = 20
    g = E // (kk * L)
    row_l = row.reshape(g, kk, L)
    et_l = edge_type.astype(jnp.int32).reshape(g, kk, L)
    el_l = edge_length.astype(F32).reshape(g, kk, L)
    rowc = row.reshape(16, E // (16 * ch), ch)
    colc = col.reshape(16, E // (16 * ch), ch)

    attr_g, attr_l = _edge_encode(
        el_l, et_l, Wg_a, bg_a.reshape(1, H), Wg_b, bg_b.reshape(1, H), emb_g,
        Wl_a, bl_a.reshape(1, H), Wl_b, bl_b.reshape(1, H), emb_l)

    node_g, node_l = _sc_scatter(attr_g, attr_l, col, N)

    gt, gb, ltab, lbot, cnt, sa = _node_transform(
        node_g, node_l, Wg1[:H], Wg1[H:], bg1.reshape(1, H),
        Wl1[:H], Wl1[H:], bl1.reshape(1, H),
        batch.astype(jnp.int32).reshape(N, 1),
        time_step.astype(jnp.int32).reshape(-1, 1),
        alphas.astype(F32).reshape(-1, 1))

    h1g, h1l = _sc_gather(gt, gb, ltab, lbot, rowc, colc, E)

    og, ol = _edge_mlp(h1g, h1l, row_l, et_l, cnt, sa,
                       Wg2, bg2.reshape(1, -1), Wg3, bg3.reshape(1, 1),
                       Wl2, bl2.reshape(1, -1), Wl3, bl3.reshape(1, 1))
    return jnp.stack([og.reshape(E), ol.reshape(E)], axis=1)


# speedup vs baseline: 4.5304x; 1.0421x over previous
import functools

import jax
import jax.numpy as jnp
from jax import lax
from jax.experimental import pallas as pl
from jax.experimental.pallas import tpu as pltpu
from jax.experimental.pallas import tpu_sc as plsc

F32 = jnp.float32


def _expand_col(xw, u, v):
    y2 = jnp.dot(u, xw, preferred_element_type=F32)
    return jnp.sum(y2 * v, axis=1, keepdims=True)


def _pack_lanes(ycol, ut, v):
    return jnp.dot(ut, ycol * v, preferred_element_type=F32)


def _enc_body(el_ref, et_ref, u_ref, v_ref,
              wga_ref, bga_ref, wgb_ref, bgb_ref, embg_ref,
              wla_ref, bla_ref, wlb_ref, blb_ref, embl_ref,
              outg_ref, outl_ref):
    elw = el_ref[0]
    etw = et_ref[0].astype(F32)
    k = elw.shape[0]
    eb = k * 128
    ncls = embg_ref.shape[0]
    u = u_ref[...]
    v = v_ref[...]
    el = _expand_col(elw, u, v)
    et = _expand_col(etw, u, v)
    ioc = lax.broadcasted_iota(jnp.int32, (eb, ncls), 1).astype(F32)
    oh = (et == ioc).astype(F32)

    def enc(wa, ba, wb, bb, emb):
        h = jnp.maximum(el * wa + ba, 0.0)
        d = jnp.dot(h, wb, preferred_element_type=F32) + bb
        return d * jnp.dot(oh, emb, preferred_element_type=F32)

    outg_ref[...] = enc(wga_ref[...], bga_ref[...], wgb_ref[...],
                        bgb_ref[...], embg_ref[...])
    outl_ref[...] = enc(wla_ref[...], bla_ref[...], wlb_ref[...],
                        blb_ref[...], embl_ref[...])


def _edge_encode(el_l, et_l, u, v, Wg_a, bg_a, Wg_b, bg_b, emb_g,
                 Wl_a, bl_a, Wl_b, bl_b, emb_l):
    G, k, L = el_l.shape
    E = G * k * L
    eb = k * L
    H = emb_g.shape[1]
    C = emb_g.shape[0]
    full = lambda s: pl.BlockSpec(s, lambda i: (0, 0))
    return pl.pallas_call(
        _enc_body,
        grid=(G,),
        in_specs=[
            pl.BlockSpec((1, k, L), lambda i: (i, 0, 0)),
            pl.BlockSpec((1, k, L), lambda i: (i, 0, 0)),
            full((eb, k)), full((eb, L)),
            full((1, H)), full((1, H)), full((H, H)), full((1, H)),
            full((C, H)),
            full((1, H)), full((1, H)), full((H, H)), full((1, H)),
            full((C, H)),
        ],
        out_specs=[pl.BlockSpec((eb, H), lambda i: (i, 0))] * 2,
        out_shape=[jax.ShapeDtypeStruct((E, H), F32)] * 2,
    )(el_l, et_l, u, v, Wg_a, bg_a, Wg_b, bg_b, emb_g,
      Wl_a, bl_a, Wl_b, bl_b, emb_l)


def _sc_scatter(attr_g, attr_l, col, N):
    E, H = attr_g.shape
    ch = 80
    info = plsc.get_sparse_core_info()
    ns = info.num_subcores
    ept = E // ns
    nch = ept // ch
    npair = nch // 2
    rpb = (N // (8 * ns)) * 8
    tail = N - rpb * ns
    nz = rpb // ch
    zrem = rpb - nz * ch
    mesh = plsc.VectorSubcoreMesh(core_axis_name="c", subcore_axis_name="s")

    @functools.partial(
        pl.kernel,
        out_type=(jax.ShapeDtypeStruct((N, H), F32),
                  jax.ShapeDtypeStruct((N, H), F32)),
        mesh=mesh,
        scratch_types=[
            pltpu.VMEM((ch,), jnp.int32),
            pltpu.VMEM((ch,), jnp.int32),
            pltpu.VMEM((2, ch, H), F32),
            pltpu.VMEM_SHARED((N, H), F32),
            pltpu.SemaphoreType.DMA,
            pltpu.SemaphoreType.DMA,
            pltpu.SemaphoreType.DMA,
        ],
    )
    def scat(attrg_h, attrl_h, col_h, outg_h, outl_h,
             idx0, idx1, rows_v, tab_s, rsem, isem, ssem):
        cid = lax.axis_index("c")
        sid = lax.axis_index("s")

        def zrow(r, carry):
            for j in range(H // 16):
                rows_v[0, r, pl.ds(j * 16, 16)] = jnp.zeros((16,), F32)
            return carry
        lax.fori_loop(0, ch, zrow, 0)

        def zcp(k, carry):
            pltpu.sync_copy(rows_v.at[0],
                            tab_s.at[pl.ds(sid * rpb + k * ch, ch)])
            return carry
        lax.fori_loop(0, nz, zcp, 0)
        if zrem:
            pltpu.sync_copy(rows_v.at[0, pl.ds(0, zrem)],
                            tab_s.at[pl.ds(sid * rpb + nz * ch, zrem)])
        if tail:
            @pl.when(sid == ns - 1)
            def _():
                pltpu.sync_copy(rows_v.at[0, pl.ds(0, tail)],
                                tab_s.at[pl.ds(ns * rpb, tail)])
        plsc.subcore_barrier()

        def run(attr_h):
            base = sid * ept
            pltpu.async_copy(col_h.at[pl.ds(base, ch)], idx0, isem)
            pltpu.async_copy(attr_h.at[pl.ds(base, ch)], rows_v.at[0], rsem)

            def wait_rows(b):
                pltpu.make_async_copy(attr_h.at[pl.ds(0, ch)],
                                      rows_v.at[b], rsem).wait()

            def wait_idx(b):
                pltpu.make_async_copy(col_h.at[pl.ds(0, ch)],
                                      idx0 if b == 0 else idx1, isem).wait()

            def drain_scat():
                pltpu.make_async_copy(attr_h.at[pl.ds(0, ch)],
                                      rows_v.at[0], ssem).wait()

            def pair(i, carry):
                for b in range(2):
                    j = 2 * i + b
                    off = base + j * ch

                    @pl.when(j >= 1)
                    def _():
                        drain_scat()

                    @pl.when(j + 1 < nch)
                    def _():
                        pltpu.async_copy(col_h.at[pl.ds(off + ch, ch)],
                                         idx1 if b == 0 else idx0, isem)
                        pltpu.async_copy(attr_h.at[pl.ds(off + ch, ch)],
                                         rows_v.at[1 - b], rsem)
                    wait_idx(b)
                    wait_rows(b)
                    pltpu.async_copy(rows_v.at[b],
                                     tab_s.at[idx0 if b == 0 else idx1],
                                     ssem, add=True)
                return carry
            lax.fori_loop(0, npair, pair, 0)
            drain_scat()

        @pl.when(cid == 0)
        def _():
            run(attrg_h)

        @pl.when(cid == 1)
        def _():
            run(attrl_h)

        plsc.subcore_barrier()

        def writeout(out_h):
            pltpu.sync_copy(tab_s.at[pl.ds(sid * rpb, rpb)],
                            out_h.at[pl.ds(sid * rpb, rpb)])
            if tail:
                @pl.when(sid == ns - 1)
                def _():
                    pltpu.sync_copy(tab_s.at[pl.ds(ns * rpb, tail)],
                                    out_h.at[pl.ds(ns * rpb, tail)])

        @pl.when(cid == 0)
        def _():
            writeout(outg_h)

        @pl.when(cid == 1)
        def _():
            writeout(outl_h)

    return scat(attr_g, attr_l, col)


def _nt_body(ng_ref, nl_ref, wgt_ref, wgbo_ref, bg1_ref,
             wlt_ref, wlbo_ref, bl1_ref, batch_ref, ts_ref, al_ref,
             gt_ref, gb_ref, lt_ref, lb_ref, cnt_ref, sa_ref):
    ng = ng_ref[...]
    nl = nl_ref[...]
    gt_ref[...] = jnp.dot(ng, wgt_ref[...], preferred_element_type=F32) + bg1_ref[...]
    gb_ref[...] = jnp.dot(ng, wgbo_ref[...], preferred_element_type=F32)
    lt_ref[...] = jnp.dot(nl, wlt_ref[...], preferred_element_type=F32) + bl1_ref[...]
    lb_ref[...] = jnp.dot(nl, wlbo_ref[...], preferred_element_type=F32)

    @pl.when(pl.program_id(0) == 0)
    def _():
        batch = batch_ref[...]
        n = batch.shape[0]
        b = cnt_ref.shape[1]
        lt = (batch < lax.broadcasted_iota(jnp.int32, (n, b), 1)).astype(F32)
        cnt_ref[...] = jnp.sum(lt, axis=0, keepdims=True)
        ts = ts_ref[...]
        t = al_ref.shape[0]
        oh = (ts == lax.broadcasted_iota(jnp.int32, (b, t), 1)).astype(F32)
        a = jnp.dot(oh, al_ref[...], preferred_element_type=F32)
        sa_ref[...] = jnp.sqrt(a) / jnp.sqrt(1.0 - a)


def _node_transform(node_g, node_l, wgt, wgb, bg1, wlt, wlb, bl1,
                    batch2, ts2, al2, nb=2000):
    N, H = node_g.shape
    B = ts2.shape[0]
    T = al2.shape[0]
    full = lambda s: pl.BlockSpec(s, lambda i: (0, 0))
    return pl.pallas_call(
        _nt_body,
        grid=(N // nb,),
        in_specs=[
            pl.BlockSpec((nb, H), lambda i: (i, 0)),
            pl.BlockSpec((nb, H), lambda i: (i, 0)),
            full((H, H)), full((H, H)), full((1, H)),
            full((H, H)), full((H, H)), full((1, H)),
            full((N, 1)), full((B, 1)), full((T, 1)),
        ],
        out_specs=[pl.BlockSpec((nb, H), lambda i: (i, 0))] * 4 + [
            full((1, B)), full((B, 1))],
        out_shape=[jax.ShapeDtypeStruct((N, H), F32)] * 4 + [
            jax.ShapeDtypeStruct((1, B), F32),
            jax.ShapeDtypeStruct((B, 1), F32)],
    )(node_g, node_l, wgt, wgb, bg1, wlt, wlb, bl1, batch2, ts2, al2)


def _sc_gather(gt, gb, ltab, lbot, row3, col3, E):
    N, H = gt.shape
    ch = row3.shape[2]
    nch = row3.shape[1]
    info = plsc.get_sparse_core_info()
    ns = info.num_subcores
    ept = E // ns
    npair = nch // 2
    mesh = plsc.VectorSubcoreMesh(core_axis_name="c", subcore_axis_name="s")

    @functools.partial(
        pl.kernel,
        out_type=(jax.ShapeDtypeStruct((E, H), F32),
                  jax.ShapeDtypeStruct((E, H), F32)),
        mesh=mesh,
        scratch_types=[
            pltpu.VMEM((nch, ch), jnp.int32),
            pltpu.VMEM((nch, ch), jnp.int32),
            pltpu.VMEM((2, ch, H), F32),
            pltpu.VMEM((2, ch, H), F32),
            pltpu.SemaphoreType.DMA,
            pltpu.SemaphoreType.DMA,
        ],
    )
    def gat(gt_h, gb_h, lt_h, lb_h, row_h, col_h, outg_h, outl_h,
            rowv, colv, av, bv, gsem, wsem):
        cid = lax.axis_index("c")
        sid = lax.axis_index("s")

        def run(t1, t2, out_h):
            pltpu.sync_copy(row_h.at[sid], rowv)
            pltpu.sync_copy(col_h.at[sid], colv)
            pltpu.async_copy(t1.at[rowv.at[0]], av.at[0], gsem)
            pltpu.async_copy(t2.at[colv.at[0]], bv.at[0], gsem)

            def wait_g(b):
                pltpu.make_async_copy(out_h.at[pl.ds(0, ch)],
                                      av.at[b], gsem).wait()
                pltpu.make_async_copy(out_h.at[pl.ds(0, ch)],
                                      bv.at[b], gsem).wait()

            def drain_w():
                pltpu.make_async_copy(out_h.at[pl.ds(0, ch)],
                                      av.at[0], wsem).wait()

            def pair(i, carry):
                for b in range(2):
                    j = 2 * i + b
                    off = sid * ept + j * ch
                    wait_g(b)

                    @pl.when(j + 1 < nch)
                    def _():
                        @pl.when(j >= 1)
                        def _():
                            drain_w()
                        pltpu.async_copy(t1.at[rowv.at[j + 1]],
                                         av.at[1 - b], gsem)
                        pltpu.async_copy(t2.at[colv.at[j + 1]],
                                         bv.at[1 - b], gsem)

                    ab = av.at[b]
                    bb = bv.at[b]

                    def add2(r, c2):
                        for rr in range(2):
                            for jj in range(H // 16):
                                sl = pl.ds(jj * 16, 16)
                                ab[2 * r + rr, sl] = (ab[2 * r + rr, sl]
                                                      + bb[2 * r + rr, sl])
                        return c2
                    lax.fori_loop(0, ch // 2, add2, 0)
                    pltpu.async_copy(ab, out_h.at[pl.ds(off, ch)], wsem)
                return carry
            lax.fori_loop(0, npair, pair, 0)
            drain_w()
            drain_w()

        @pl.when(cid == 0)
        def _():
            run(gt_h, gb_h, outg_h)

        @pl.when(cid == 1)
        def _():
            run(lt_h, lb_h, outl_h)

    return gat(gt, gb, ltab, lbot, row3, col3)


def _mlp_body(h1g_ref, h1l_ref, roww_ref, etw_ref, u_ref, ut_ref, v_ref,
              cnt_ref, sa_ref,
              wg2_ref, bg2_ref, wg3_ref, bg3_ref,
              wl2_ref, bl2_ref, wl3_ref, bl3_ref, og_ref, ol_ref):
    def mlp(x, w2, b2, w3, b3):
        h = jnp.maximum(jnp.dot(jnp.maximum(x, 0.0), w2,
                                preferred_element_type=F32) + b2, 0.0)
        return jnp.dot(h, w3, preferred_element_type=F32) + b3

    og = mlp(h1g_ref[...], wg2_ref[...], bg2_ref[...], wg3_ref[...],
             bg3_ref[...])
    ol = mlp(h1l_ref[...], wl2_ref[...], bl2_ref[...], wl3_ref[...],
             bl3_ref[...])
    roww = roww_ref[0].astype(F32)
    etw = etw_ref[0]
    k = roww.shape[0]
    eb = k * 128
    b = cnt_ref.shape[1]
    u = u_ref[...]
    ut = ut_ref[...]
    v = v_ref[...]
    row = _expand_col(roww, u, v)
    cnt = cnt_ref[...]
    ge = (row >= cnt).astype(F32)
    e2g = jnp.sum(ge, axis=1, keepdims=True) - 1.0
    iob = lax.broadcasted_iota(jnp.int32, (eb, b), 1).astype(F32)
    oh = (e2g == iob).astype(F32)
    scale = jnp.dot(oh, sa_ref[...], preferred_element_type=F32)
    maskw = (etw > 0).astype(F32)
    og_ref[0] = _pack_lanes(og * scale, ut, v)
    ol_ref[0] = _pack_lanes(ol, ut, v) * maskw


def _edge_mlp(h1g, h1l, row_l, et_l, u, ut, v, cnt, sa,
              Wg2, bg2, Wg3, bg3, Wl2, bl2, Wl3, bl3):
    E, H = h1g.shape
    G, k, L = row_l.shape
    eb = k * L
    B = cnt.shape[1]
    K = Wg2.shape[1]
    full = lambda s: pl.BlockSpec(s, lambda i: (0, 0))
    return pl.pallas_call(
        _mlp_body,
        grid=(G,),
        in_specs=[
            pl.BlockSpec((eb, H), lambda i: (i, 0)),
            pl.BlockSpec((eb, H), lambda i: (i, 0)),
            pl.BlockSpec((1, k, L), lambda i: (i, 0, 0)),
            pl.BlockSpec((1, k, L), lambda i: (i, 0, 0)),
            full((eb, k)), full((k, eb)), full((eb, L)),
            full((1, B)), full((B, 1)),
            full((H, K)), full((1, K)), full((K, 1)), full((1, 1)),
            full((H, K)), full((1, K)), full((K, 1)), full((1, 1)),
        ],
        out_specs=[pl.BlockSpec((1, k, L), lambda i: (i, 0, 0))] * 2,
        out_shape=[jax.ShapeDtypeStruct((G, k, L), F32)] * 2,
    )(h1g, h1l, row_l, et_l, u, ut, v, cnt, sa,
      Wg2, bg2, Wg3, bg3, Wl2, bl2, Wl3, bl3)


def kernel(atom_type, pos, bond_index, bond_type, batch, time_step,
           edge_index, edge_type, edge_length, alphas,
           emb_g, Wg_a, bg_a, Wg_b, bg_b, emb_l, Wl_a, bl_a, Wl_b, bl_b,
           Wg1, bg1, Wg2, bg2, Wg3, bg3, Wl1, bl1, Wl2, bl2, Wl3, bl3):
    E = edge_index.shape[1]
    N = batch.shape[0]
    H = emb_g.shape[1]
    ch = 80
    L = 128
    row = edge_index[0].astype(jnp.int32)
    col = edge_index[1].astype(jnp.int32)
    kk = 20
    g = E // (kk * L)
    eb = kk * L
    u = (jnp.arange(eb)[:, None] // L ==
         jnp.arange(kk)[None, :]).astype(F32)
    v = (jnp.arange(eb)[:, None] % L ==
         jnp.arange(L)[None, :]).astype(F32)
    ut = u.T
    row_l = row.reshape(g, kk, L)
    et_l = edge_type.astype(jnp.int32).reshape(g, kk, L)
    el_l = edge_length.astype(F32).reshape(g, kk, L)
    rowc = row.reshape(16, E // (16 * ch), ch)
    colc = col.reshape(16, E // (16 * ch), ch)

    attr_g, attr_l = _edge_encode(
        el_l, et_l, u, v, Wg_a, bg_a.reshape(1, H), Wg_b, bg_b.reshape(1, H), emb_g,
        Wl_a, bl_a.reshape(1, H), Wl_b, bl_b.reshape(1, H), emb_l)

    node_g, node_l = _sc_scatter(attr_g, attr_l, col, N)

    gt, gb, ltab, lbot, cnt, sa = _node_transform(
        node_g, node_l, Wg1[:H], Wg1[H:], bg1.reshape(1, H),
        Wl1[:H], Wl1[H:], bl1.reshape(1, H),
        batch.astype(jnp.int32).reshape(N, 1),
        time_step.astype(jnp.int32).reshape(-1, 1),
        alphas.astype(F32).reshape(-1, 1))

    h1g, h1l = _sc_gather(gt, gb, ltab, lbot, rowc, colc, E)

    og, ol = _edge_mlp(h1g, h1l, row_l, et_l, u, ut, v, cnt, sa,
                       Wg2, bg2.reshape(1, -1), Wg3, bg3.reshape(1, 1),
                       Wl2, bl2.reshape(1, -1), Wl3, bl3.reshape(1, 1))
    return jnp.stack([og.reshape(E), ol.reshape(E)], axis=1)


# speedup vs baseline: 4.8759x; 1.0762x over previous
import functools

import jax
import jax.numpy as jnp
from jax import lax
from jax.experimental import pallas as pl
from jax.experimental.pallas import tpu as pltpu
from jax.experimental.pallas import tpu_sc as plsc

F32 = jnp.float32


def _expand_col(xw, u, v):
    y2 = jnp.dot(u, xw, preferred_element_type=F32)
    return jnp.sum(y2 * v, axis=1, keepdims=True)


def _pack_lanes(ycol, ut, v):
    return jnp.dot(ut, ycol * v, preferred_element_type=F32)


def _enc_body(el_ref, et_ref, u_ref, v_ref,
              wga_ref, bga_ref, wgb_ref, bgb_ref, embg_ref,
              wla_ref, bla_ref, wlb_ref, blb_ref, embl_ref,
              outg_ref, outl_ref):
    elw = el_ref[0]
    etw = et_ref[0].astype(F32)
    k = elw.shape[0]
    eb = k * 128
    ncls = embg_ref.shape[0]
    u = u_ref[...]
    v = v_ref[...]
    el = _expand_col(elw, u, v)
    et = _expand_col(etw, u, v)
    ioc = lax.broadcasted_iota(jnp.int32, (eb, ncls), 1).astype(F32)
    oh = (et == ioc).astype(F32)

    def enc(wa, ba, wb, bb, emb):
        h = jnp.maximum(el * wa + ba, 0.0)
        d = jnp.dot(h, wb, preferred_element_type=F32) + bb
        return d * jnp.dot(oh, emb, preferred_element_type=F32)

    outg_ref[...] = enc(wga_ref[...], bga_ref[...], wgb_ref[...],
                        bgb_ref[...], embg_ref[...])
    outl_ref[...] = enc(wla_ref[...], bla_ref[...], wlb_ref[...],
                        blb_ref[...], embl_ref[...])


def _edge_encode(el_l, et_l, u, v, Wg_a, bg_a, Wg_b, bg_b, emb_g,
                 Wl_a, bl_a, Wl_b, bl_b, emb_l):
    G, k, L = el_l.shape
    E = G * k * L
    eb = k * L
    H = emb_g.shape[1]
    C = emb_g.shape[0]
    full = lambda s: pl.BlockSpec(s, lambda i: (0, 0))
    return pl.pallas_call(
        _enc_body,
        grid=(G,),
        in_specs=[
            pl.BlockSpec((1, k, L), lambda i: (i, 0, 0)),
            pl.BlockSpec((1, k, L), lambda i: (i, 0, 0)),
            full((eb, k)), full((eb, L)),
            full((1, H)), full((1, H)), full((H, H)), full((1, H)),
            full((C, H)),
            full((1, H)), full((1, H)), full((H, H)), full((1, H)),
            full((C, H)),
        ],
        out_specs=[pl.BlockSpec((eb, H), lambda i: (i, 0))] * 2,
        out_shape=[jax.ShapeDtypeStruct((E, H), F32)] * 2,
    )(el_l, et_l, u, v, Wg_a, bg_a, Wg_b, bg_b, emb_g,
      Wl_a, bl_a, Wl_b, bl_b, emb_l)


def _sc_scatter(attr_g, attr_l, col, N):
    E, H = attr_g.shape
    ch = 80
    info = plsc.get_sparse_core_info()
    ns = info.num_subcores
    ept = E // ns
    nch = ept // ch
    npair = nch // 2
    rpb = (N // (8 * ns)) * 8
    tail = N - rpb * ns
    nz = rpb // ch
    zrem = rpb - nz * ch
    mesh = plsc.VectorSubcoreMesh(core_axis_name="c", subcore_axis_name="s")

    @functools.partial(
        pl.kernel,
        out_type=(jax.ShapeDtypeStruct((N, H), F32),
                  jax.ShapeDtypeStruct((N, H), F32)),
        mesh=mesh,
        scratch_types=[
            pltpu.VMEM((ch,), jnp.int32),
            pltpu.VMEM((ch,), jnp.int32),
            pltpu.VMEM((2, ch, H), F32),
            pltpu.VMEM_SHARED((N, H), F32),
            pltpu.SemaphoreType.DMA,
            pltpu.SemaphoreType.DMA,
            pltpu.SemaphoreType.DMA,
        ],
    )
    def scat(attrg_h, attrl_h, col_h, outg_h, outl_h,
             idx0, idx1, rows_v, tab_s, rsem, isem, ssem):
        cid = lax.axis_index("c")
        sid = lax.axis_index("s")

        def zrow(r, carry):
            for j in range(H // 16):
                rows_v[0, r, pl.ds(j * 16, 16)] = jnp.zeros((16,), F32)
            return carry
        lax.fori_loop(0, ch, zrow, 0)

        def zcp(k, carry):
            pltpu.sync_copy(rows_v.at[0],
                            tab_s.at[pl.ds(sid * rpb + k * ch, ch)])
            return carry
        lax.fori_loop(0, nz, zcp, 0)
        if zrem:
            pltpu.sync_copy(rows_v.at[0, pl.ds(0, zrem)],
                            tab_s.at[pl.ds(sid * rpb + nz * ch, zrem)])
        if tail:
            @pl.when(sid == ns - 1)
            def _():
                pltpu.sync_copy(rows_v.at[0, pl.ds(0, tail)],
                                tab_s.at[pl.ds(ns * rpb, tail)])
        plsc.subcore_barrier()

        def run(attr_h):
            base = sid * ept
            pltpu.async_copy(col_h.at[pl.ds(base, ch)], idx0, isem)
            pltpu.async_copy(attr_h.at[pl.ds(base, ch)], rows_v.at[0], rsem)

            def wait_rows(b):
                pltpu.make_async_copy(attr_h.at[pl.ds(0, ch)],
                                      rows_v.at[b], rsem).wait()

            def wait_idx(b):
                pltpu.make_async_copy(col_h.at[pl.ds(0, ch)],
                                      idx0 if b == 0 else idx1, isem).wait()

            def drain_scat():
                pltpu.make_async_copy(attr_h.at[pl.ds(0, ch)],
                                      rows_v.at[0], ssem).wait()

            def pair(i, carry):
                for b in range(2):
                    j = 2 * i + b
                    off = base + j * ch

                    @pl.when(j >= 1)
                    def _():
                        drain_scat()

                    @pl.when(j + 1 < nch)
                    def _():
                        pltpu.async_copy(col_h.at[pl.ds(off + ch, ch)],
                                         idx1 if b == 0 else idx0, isem)
                        pltpu.async_copy(attr_h.at[pl.ds(off + ch, ch)],
                                         rows_v.at[1 - b], rsem)
                    wait_idx(b)
                    wait_rows(b)
                    pltpu.async_copy(rows_v.at[b],
                                     tab_s.at[idx0 if b == 0 else idx1],
                                     ssem, add=True)
                return carry
            lax.fori_loop(0, npair, pair, 0)
            drain_scat()

        @pl.when(cid == 0)
        def _():
            run(attrg_h)

        @pl.when(cid == 1)
        def _():
            run(attrl_h)

        plsc.subcore_barrier()

        def writeout(out_h):
            pltpu.sync_copy(tab_s.at[pl.ds(sid * rpb, rpb)],
                            out_h.at[pl.ds(sid * rpb, rpb)])
            if tail:
                @pl.when(sid == ns - 1)
                def _():
                    pltpu.sync_copy(tab_s.at[pl.ds(ns * rpb, tail)],
                                    out_h.at[pl.ds(ns * rpb, tail)])

        @pl.when(cid == 0)
        def _():
            writeout(outg_h)

        @pl.when(cid == 1)
        def _():
            writeout(outl_h)

    return scat(attr_g, attr_l, col)


def _nt_body(ng_ref, nl_ref, wgt_ref, wgbo_ref, bg1_ref,
             wlt_ref, wlbo_ref, bl1_ref, batch_ref, ts_ref, al_ref,
             gt_ref, gb_ref, lt_ref, lb_ref, cnt_ref, sa_ref):
    ng = ng_ref[...]
    nl = nl_ref[...]
    gt_ref[...] = jnp.dot(ng, wgt_ref[...], preferred_element_type=F32) + bg1_ref[...]
    gb_ref[...] = jnp.dot(ng, wgbo_ref[...], preferred_element_type=F32)
    lt_ref[...] = jnp.dot(nl, wlt_ref[...], preferred_element_type=F32) + bl1_ref[...]
    lb_ref[...] = jnp.dot(nl, wlbo_ref[...], preferred_element_type=F32)

    @pl.when(pl.program_id(0) == 0)
    def _():
        batch = batch_ref[...]
        n = batch.shape[0]
        b = cnt_ref.shape[1]
        lt = (batch < lax.broadcasted_iota(jnp.int32, (n, b), 1)).astype(F32)
        cnt_ref[...] = jnp.sum(lt, axis=0, keepdims=True)
        ts = ts_ref[...]
        t = al_ref.shape[0]
        oh = (ts == lax.broadcasted_iota(jnp.int32, (b, t), 1)).astype(F32)
        a = jnp.dot(oh, al_ref[...], preferred_element_type=F32)
        sa = jnp.sqrt(a) / jnp.sqrt(1.0 - a)
        dsa = sa - jnp.concatenate([jnp.zeros((1, 1), F32), sa[:-1]], axis=0)
        sa_ref[...] = dsa


def _node_transform(node_g, node_l, wgt, wgb, bg1, wlt, wlb, bl1,
                    batch2, ts2, al2, nb=2000):
    N, H = node_g.shape
    B = ts2.shape[0]
    T = al2.shape[0]
    full = lambda s: pl.BlockSpec(s, lambda i: (0, 0))
    return pl.pallas_call(
        _nt_body,
        grid=(N // nb,),
        in_specs=[
            pl.BlockSpec((nb, H), lambda i: (i, 0)),
            pl.BlockSpec((nb, H), lambda i: (i, 0)),
            full((H, H)), full((H, H)), full((1, H)),
            full((H, H)), full((H, H)), full((1, H)),
            full((N, 1)), full((B, 1)), full((T, 1)),
        ],
        out_specs=[pl.BlockSpec((nb, H), lambda i: (i, 0))] * 4 + [
            full((1, B)), full((B, 1))],
        out_shape=[jax.ShapeDtypeStruct((N, H), F32)] * 4 + [
            jax.ShapeDtypeStruct((1, B), F32),
            jax.ShapeDtypeStruct((B, 1), F32)],
    )(node_g, node_l, wgt, wgb, bg1, wlt, wlb, bl1, batch2, ts2, al2)


def _sc_gather(gt, gb, ltab, lbot, row3, col3, E):
    N, H = gt.shape
    ch = row3.shape[2]
    nch = row3.shape[1]
    info = plsc.get_sparse_core_info()
    ns = info.num_subcores
    ept = E // ns
    npair = nch // 2
    mesh = plsc.VectorSubcoreMesh(core_axis_name="c", subcore_axis_name="s")

    @functools.partial(
        pl.kernel,
        out_type=(jax.ShapeDtypeStruct((E, H), F32),
                  jax.ShapeDtypeStruct((E, H), F32)),
        mesh=mesh,
        scratch_types=[
            pltpu.VMEM((nch, ch), jnp.int32),
            pltpu.VMEM((nch, ch), jnp.int32),
            pltpu.VMEM((2, ch, H), F32),
            pltpu.VMEM((2, ch, H), F32),
            pltpu.SemaphoreType.DMA,
            pltpu.SemaphoreType.DMA,
        ],
    )
    def gat(gt_h, gb_h, lt_h, lb_h, row_h, col_h, outg_h, outl_h,
            rowv, colv, av, bv, gsem, wsem):
        cid = lax.axis_index("c")
        sid = lax.axis_index("s")

        def run(t1, t2, out_h):
            pltpu.sync_copy(row_h.at[sid], rowv)
            pltpu.sync_copy(col_h.at[sid], colv)
            pltpu.async_copy(t1.at[rowv.at[0]], av.at[0], gsem)
            pltpu.async_copy(t2.at[colv.at[0]], bv.at[0], gsem)

            def wait_g(b):
                pltpu.make_async_copy(out_h.at[pl.ds(0, ch)],
                                      av.at[b], gsem).wait()
                pltpu.make_async_copy(out_h.at[pl.ds(0, ch)],
                                      bv.at[b], gsem).wait()

            def drain_w():
                pltpu.make_async_copy(out_h.at[pl.ds(0, ch)],
                                      av.at[0], wsem).wait()

            def pair(i, carry):
                for b in range(2):
                    j = 2 * i + b
                    off = sid * ept + j * ch
                    wait_g(b)

                    @pl.when(j + 1 < nch)
                    def _():
                        @pl.when(j >= 1)
                        def _():
                            drain_w()
                        pltpu.async_copy(t1.at[rowv.at[j + 1]],
                                         av.at[1 - b], gsem)
                        pltpu.async_copy(t2.at[colv.at[j + 1]],
                                         bv.at[1 - b], gsem)

                    ab = av.at[b]
                    bb = bv.at[b]

                    def add2(r, c2):
                        for rr in range(2):
                            for jj in range(H // 16):
                                sl = pl.ds(jj * 16, 16)
                                ab[2 * r + rr, sl] = (ab[2 * r + rr, sl]
                                                      + bb[2 * r + rr, sl])
                        return c2
                    lax.fori_loop(0, ch // 2, add2, 0)
                    pltpu.async_copy(ab, out_h.at[pl.ds(off, ch)], wsem)
                return carry
            lax.fori_loop(0, npair, pair, 0)
            drain_w()
            drain_w()

        @pl.when(cid == 0)
        def _():
            run(gt_h, gb_h, outg_h)

        @pl.when(cid == 1)
        def _():
            run(lt_h, lb_h, outl_h)

    return gat(gt, gb, ltab, lbot, row3, col3)


def _mlp_body(h1g_ref, h1l_ref, roww_ref, etw_ref, u_ref, ut_ref, v_ref,
              cnt_ref, dsa_ref,
              wg2_ref, bg2_ref, wl2_ref, bl2_ref, w3_ref, b3_ref,
              og_ref, ol_ref):
    hg = jnp.maximum(jnp.dot(jnp.maximum(h1g_ref[...], 0.0), wg2_ref[...],
                             preferred_element_type=F32) + bg2_ref[...], 0.0)
    hl = jnp.maximum(jnp.dot(jnp.maximum(h1l_ref[...], 0.0), wl2_ref[...],
                             preferred_element_type=F32) + bl2_ref[...], 0.0)
    h2 = jnp.concatenate([hg, hl], axis=1)
    out2 = jnp.dot(h2, w3_ref[...], preferred_element_type=F32) + b3_ref[...]
    og = out2[:, 0:1]
    ol = out2[:, 1:2]
    roww = roww_ref[0].astype(F32)
    etw = etw_ref[0]
    u = u_ref[...]
    ut = ut_ref[...]
    v = v_ref[...]
    row = _expand_col(roww, u, v)
    cnt = cnt_ref[...]
    ge = (row >= cnt).astype(F32)
    scale = jnp.dot(ge, dsa_ref[...], preferred_element_type=F32)
    maskw = (etw > 0).astype(F32)
    og_ref[0] = _pack_lanes(og * scale, ut, v)
    ol_ref[0] = _pack_lanes(ol, ut, v) * maskw


def _edge_mlp(h1g, h1l, row_l, et_l, u, ut, v, cnt, dsa,
              Wg2, bg2, Wl2, bl2, w3cat, b3cat):
    E, H = h1g.shape
    G, k, L = row_l.shape
    eb = k * L
    B = cnt.shape[1]
    K = Wg2.shape[1]
    full = lambda s: pl.BlockSpec(s, lambda i: (0, 0))
    return pl.pallas_call(
        _mlp_body,
        grid=(G,),
        in_specs=[
            pl.BlockSpec((eb, H), lambda i: (i, 0)),
            pl.BlockSpec((eb, H), lambda i: (i, 0)),
            pl.BlockSpec((1, k, L), lambda i: (i, 0, 0)),
            pl.BlockSpec((1, k, L), lambda i: (i, 0, 0)),
            full((eb, k)), full((k, eb)), full((eb, L)),
            full((1, B)), full((B, 1)),
            full((H, K)), full((1, K)),
            full((H, K)), full((1, K)),
            full((2 * K, 2)), full((1, 2)),
        ],
        out_specs=[pl.BlockSpec((1, k, L), lambda i: (i, 0, 0))] * 2,
        out_shape=[jax.ShapeDtypeStruct((G, k, L), F32)] * 2,
    )(h1g, h1l, row_l, et_l, u, ut, v, cnt, dsa,
      Wg2, bg2, Wl2, bl2, w3cat, b3cat)


def kernel(atom_type, pos, bond_index, bond_type, batch, time_step,
           edge_index, edge_type, edge_length, alphas,
           emb_g, Wg_a, bg_a, Wg_b, bg_b, emb_l, Wl_a, bl_a, Wl_b, bl_b,
           Wg1, bg1, Wg2, bg2, Wg3, bg3, Wl1, bl1, Wl2, bl2, Wl3, bl3):
    E = edge_index.shape[1]
    N = batch.shape[0]
    H = emb_g.shape[1]
    ch = 80
    L = 128
    row = edge_index[0].astype(jnp.int32)
    col = edge_index[1].astype(jnp.int32)
    kk = 50
    g = E // (kk * L)
    eb = kk * L
    u = (jnp.arange(eb)[:, None] // L ==
         jnp.arange(kk)[None, :]).astype(F32)
    v = (jnp.arange(eb)[:, None] % L ==
         jnp.arange(L)[None, :]).astype(F32)
    ut = u.T
    row_l = row.reshape(g, kk, L)
    et_l = edge_type.astype(jnp.int32).reshape(g, kk, L)
    el_l = edge_length.astype(F32).reshape(g, kk, L)
    rowc = row.reshape(16, E // (16 * ch), ch)
    colc = col.reshape(16, E // (16 * ch), ch)

    attr_g, attr_l = _edge_encode(
        el_l, et_l, u, v, Wg_a, bg_a.reshape(1, H), Wg_b, bg_b.reshape(1, H), emb_g,
        Wl_a, bl_a.reshape(1, H), Wl_b, bl_b.reshape(1, H), emb_l)

    node_g, node_l = _sc_scatter(attr_g, attr_l, col, N)

    gt, gb, ltab, lbot, cnt, sa = _node_transform(
        node_g, node_l, Wg1[:H], Wg1[H:], bg1.reshape(1, H),
        Wl1[:H], Wl1[H:], bl1.reshape(1, H),
        batch.astype(jnp.int32).reshape(N, 1),
        time_step.astype(jnp.int32).reshape(-1, 1),
        alphas.astype(F32).reshape(-1, 1))

    h1g, h1l = _sc_gather(gt, gb, ltab, lbot, rowc, colc, E)

    z3 = jnp.zeros_like(Wg3)
    w3cat = jnp.concatenate(
        [jnp.concatenate([Wg3, z3], axis=1),
         jnp.concatenate([z3, Wl3], axis=1)], axis=0)
    b3cat = jnp.stack([bg3[0], bl3[0]]).reshape(1, 2)
    og, ol = _edge_mlp(h1g, h1l, row_l, et_l, u, ut, v, cnt, sa,
                       Wg2, bg2.reshape(1, -1), Wl2, bl2.reshape(1, -1),
                       w3cat, b3cat)
    return jnp.stack([og.reshape(E), ol.reshape(E)], axis=1)
